# initial kernel scaffold (unmeasured)
import jax
import jax.numpy as jnp
from jax import lax
from jax.experimental import pallas as pl
from jax.experimental.pallas import tpu as pltpu

M = 2048
K = 4096
NV = 8192

BM = 256
BN = 512
MBLK = M // BM
NBLK = NV // BN

CM = 256
CBLK = M // CM


def _gemm_body(x_ref, w_ref, e_ref, s_ref):
    acc = jnp.dot(x_ref[...], w_ref[...], preferred_element_type=jnp.float32)
    e = jnp.exp(acc)
    e_ref[...] = e
    s_ref[...] = jnp.sum(e, axis=1, keepdims=True)


def _gemm_exp(x, W):
    return pl.pallas_call(
        _gemm_body,
        grid=(MBLK, NBLK),
        in_specs=[
            pl.BlockSpec((BM, K), lambda i, j: (i, 0)),
            pl.BlockSpec((K, BN), lambda i, j: (0, j)),
        ],
        out_specs=[
            pl.BlockSpec((BM, BN), lambda i, j: (i, j)),
            pl.BlockSpec((BM, 1), lambda i, j: (i, j)),
        ],
        out_shape=[
            jax.ShapeDtypeStruct((M, NV), jnp.float32),
            jax.ShapeDtypeStruct((M, NBLK), jnp.float32),
        ],
    )(x, W)


def _comm_body(e_ref, s_ref, out_ref, buf, s_nbr, recip,
               send_sems, recv_sems, store_sems, load_sems,
               s_send, s_recv):
    mx = lax.axis_index("x")
    my = lax.axis_index("y")
    mz = lax.axis_index("z")
    nbr = (mx, 1 - my, mz)

    barrier = pltpu.get_barrier_semaphore()
    pl.semaphore_signal(barrier, inc=1, device_id=nbr,
                        device_id_type=pl.DeviceIdType.MESH)
    pl.semaphore_wait(barrier, 1)

    s_rdma = pltpu.make_async_remote_copy(
        src_ref=s_ref, dst_ref=s_nbr, send_sem=s_send, recv_sem=s_recv,
        device_id=nbr, device_id_type=pl.DeviceIdType.MESH)
    s_rdma.start()
    s_rdma.wait()
    recip[...] = 1.0 / (s_ref[...] + s_nbr[...])

    my_col = my * NV
    nbr_col = (1 - my) * NV

    def tile_send(i, slot):
        return pltpu.make_async_remote_copy(
            src_ref=buf.at[slot],
            dst_ref=out_ref.at[pl.ds(i * CM, CM), pl.ds(my_col, NV)],
            send_sem=send_sems.at[i],
            recv_sem=recv_sems.at[i],
            device_id=nbr, device_id_type=pl.DeviceIdType.MESH)

    def tile_store(i, slot):
        return pltpu.make_async_copy(
            buf.at[slot],
            out_ref.at[pl.ds(i * CM, CM), pl.ds(my_col, NV)],
            store_sems.at[i])

    for i in range(CBLK):
        slot = i % 2
        if i >= 2:
            tile_send(i - 2, slot).wait_send()
            tile_store(i - 2, slot).wait()
        ld = pltpu.make_async_copy(
            e_ref.at[pl.ds(i * CM, CM), :], buf.at[slot], load_sems.at[slot])
        ld.start()
        ld.wait()
        buf[slot] = buf[slot] * recip[pl.ds(i * CM, CM), :]
        tile_send(i, slot).start()
        tile_store(i, slot).start()

    for i in range(max(0, CBLK - 2), CBLK):
        slot = i % 2
        tile_send(i, slot).wait_send()
        tile_store(i, slot).wait()

    for i in range(CBLK):
        pltpu.make_async_remote_copy(
            src_ref=buf.at[0],
            dst_ref=out_ref.at[pl.ds(i * CM, CM), pl.ds(nbr_col, NV)],
            send_sem=send_sems.at[i],
            recv_sem=recv_sems.at[i],
            device_id=nbr, device_id_type=pl.DeviceIdType.MESH,
        ).wait_recv()


def _comm_assemble(e, s):
    return pl.pallas_call(
        _comm_body,
        in_specs=[
            pl.BlockSpec(memory_space=pl.ANY),
            pl.BlockSpec(memory_space=pltpu.MemorySpace.VMEM),
        ],
        out_specs=pl.BlockSpec(memory_space=pl.ANY),
        out_shape=jax.ShapeDtypeStruct((M, 2 * NV), jnp.float32),
        scratch_shapes=[
            pltpu.MemorySpace.VMEM((2, CM, NV), jnp.float32),
            pltpu.MemorySpace.VMEM((M, 1), jnp.float32),
            pltpu.MemorySpace.VMEM((M, 1), jnp.float32),
            pltpu.SemaphoreType.DMA((CBLK,)),
            pltpu.SemaphoreType.DMA((CBLK,)),
            pltpu.SemaphoreType.DMA((CBLK,)),
            pltpu.SemaphoreType.DMA((2,)),
            pltpu.SemaphoreType.DMA,
            pltpu.SemaphoreType.DMA,
        ],
        compiler_params=pltpu.CompilerParams(collective_id=0),
    )(e, s)


def kernel(x, W):
    e, s_part = _gemm_exp(x, W)
    s = s_part.sum(axis=1, keepdims=True)
    return _comm_assemble(e, s)


# baseline (device time: 1238997 ns/iter reference)
import jax
import jax.numpy as jnp
from jax import lax
from jax.experimental import pallas as pl
from jax.experimental.pallas import tpu as pltpu

M = 2048
K = 4096
NV = 8192

BM = 256
BN = 512
MBLK = M // BM
NBLK = NV // BN

CM = 256
CBLK = M // CM


def _gemm_body(x_ref, w_ref, e_ref, s_ref):
    acc = jnp.dot(x_ref[...], w_ref[...], preferred_element_type=jnp.float32)
    e = jnp.exp(acc)
    e_ref[...] = e

    @pl.when(pl.program_id(1) == 0)
    def _():
        s_ref[...] = jnp.zeros_like(s_ref)

    s_ref[:, 0:1] = s_ref[:, 0:1] + jnp.sum(e, axis=1, keepdims=True)


def _gemm_exp(x, W):
    return pl.pallas_call(
        _gemm_body,
        grid=(MBLK, NBLK),
        in_specs=[
            pl.BlockSpec((BM, K), lambda i, j: (i, 0)),
            pl.BlockSpec((K, BN), lambda i, j: (0, j)),
        ],
        out_specs=[
            pl.BlockSpec((BM, BN), lambda i, j: (i, j)),
            pl.BlockSpec((BM, 128), lambda i, j: (i, 0)),
        ],
        out_shape=[
            jax.ShapeDtypeStruct((M, NV), jnp.float32),
            jax.ShapeDtypeStruct((M, 128), jnp.float32),
        ],
    )(x, W)


def _comm_body(e_ref, s_ref, out_ref, buf, s_nbr, recip,
               send_sems, recv_sems, store_sems, load_sems,
               s_send, s_recv):
    mx = lax.axis_index("x")
    my = lax.axis_index("y")
    mz = lax.axis_index("z")
    nbr = (mx, 1 - my, mz)

    barrier = pltpu.get_barrier_semaphore()
    pl.semaphore_signal(barrier, inc=1, device_id=nbr,
                        device_id_type=pl.DeviceIdType.MESH)
    pl.semaphore_wait(barrier, 1)

    s_rdma = pltpu.make_async_remote_copy(
        src_ref=s_ref, dst_ref=s_nbr, send_sem=s_send, recv_sem=s_recv,
        device_id=nbr, device_id_type=pl.DeviceIdType.MESH)
    s_rdma.start()
    s_rdma.wait()
    recip[...] = 1.0 / (s_ref[...] + s_nbr[...])

    for my_val in (0, 1):
        @pl.when(my == my_val)
        def _(my_val=my_val):
            my_col = my_val * NV
            nbr_col = (1 - my_val) * NV

            for i in range(CBLK):
                slot = i % 2
                ld = pltpu.make_async_copy(
                    e_ref.at[pl.ds(i * CM, CM), :], buf.at[slot],
                    load_sems.at[slot])
                ld.start()
                ld.wait()
                buf[slot] = buf[slot] * recip[pl.ds(i * CM, CM), :]
                rdma = pltpu.make_async_remote_copy(
                    src_ref=buf.at[slot],
                    dst_ref=out_ref.at[pl.ds(i * CM, CM), pl.ds(my_col, NV)],
                    send_sem=send_sems.at[i], recv_sem=recv_sems.at[i],
                    device_id=nbr, device_id_type=pl.DeviceIdType.MESH)
                rdma.start()
                rdma.wait_send()
                st = pltpu.make_async_copy(
                    buf.at[slot],
                    out_ref.at[pl.ds(i * CM, CM), pl.ds(my_col, NV)],
                    store_sems.at[i])
                st.start()
                st.wait()

            for i in range(CBLK):
                pltpu.make_async_remote_copy(
                    src_ref=buf.at[0],
                    dst_ref=out_ref.at[pl.ds(i * CM, CM), pl.ds(nbr_col, NV)],
                    send_sem=send_sems.at[i],
                    recv_sem=recv_sems.at[i],
                    device_id=nbr, device_id_type=pl.DeviceIdType.MESH,
                ).wait_recv()


def _comm_assemble(e, s):
    return pl.pallas_call(
        _comm_body,
        in_specs=[
            pl.BlockSpec(memory_space=pl.ANY),
            pl.BlockSpec(memory_space=pltpu.MemorySpace.VMEM),
        ],
        out_specs=pl.BlockSpec(memory_space=pl.ANY),
        out_shape=jax.ShapeDtypeStruct((M, 2 * NV), jnp.float32),
        scratch_shapes=[
            pltpu.MemorySpace.VMEM((2, CM, NV), jnp.float32),
            pltpu.MemorySpace.VMEM((M, 1), jnp.float32),
            pltpu.MemorySpace.VMEM((M, 1), jnp.float32),
            pltpu.SemaphoreType.DMA((CBLK,)),
            pltpu.SemaphoreType.DMA((CBLK,)),
            pltpu.SemaphoreType.DMA((CBLK,)),
            pltpu.SemaphoreType.DMA((2,)),
            pltpu.SemaphoreType.DMA,
            pltpu.SemaphoreType.DMA,
        ],
        compiler_params=pltpu.CompilerParams(collective_id=0),
    )(e, s)


def kernel(x, W):
    e, s_part = _gemm_exp(x, W)
    s = s_part[:, 0:1]
    return _comm_assemble(e, s)


# device time: 739874 ns/iter; 1.6746x vs baseline; 1.6746x over previous
import jax
import jax.numpy as jnp
from jax import lax
from jax.experimental import pallas as pl
from jax.experimental.pallas import tpu as pltpu

M = 2048
K = 4096
NV = 8192

BM = 512
BN = 512
MBLK = M // BM
NBLK = NV // BN

CM = 256
CBLK = M // CM

MESH = pl.DeviceIdType.MESH


def _gemm_body(x_ref, w_ref, e_ref, s_ref):
    acc = jnp.dot(x_ref[...], w_ref[...], preferred_element_type=jnp.float32)
    e = jnp.exp(acc)
    e_ref[...] = e.astype(jnp.bfloat16)

    @pl.when(pl.program_id(1) == 0)
    def _():
        s_ref[...] = jnp.zeros_like(s_ref)

    s_ref[:, 0:1] = s_ref[:, 0:1] + jnp.sum(e, axis=1, keepdims=True)


def _gemm_exp(x16, W16):
    return pl.pallas_call(
        _gemm_body,
        grid=(MBLK, NBLK),
        in_specs=[
            pl.BlockSpec((BM, K), lambda i, j: (i, 0)),
            pl.BlockSpec((K, BN), lambda i, j: (0, j)),
        ],
        out_specs=[
            pl.BlockSpec((BM, BN), lambda i, j: (i, j)),
            pl.BlockSpec((BM, 128), lambda i, j: (i, 0)),
        ],
        out_shape=[
            jax.ShapeDtypeStruct((M, NV), jnp.bfloat16),
            jax.ShapeDtypeStruct((M, 128), jnp.float32),
        ],
    )(x16, W16)


def _comm_body(e_ref, s_ref, out_ref, recv_ref, buf, rbuf, st_buf,
               s_nbr, recip_ref,
               send_sems, recv_sems, store_sems, load_sem, rload_sem,
               s_send, s_recv):
    mx = lax.axis_index("x")
    my = lax.axis_index("y")
    mz = lax.axis_index("z")
    nbr = (mx, 1 - my, mz)

    barrier = pltpu.get_barrier_semaphore()
    pl.semaphore_signal(barrier, inc=1, device_id=nbr, device_id_type=MESH)
    pl.semaphore_wait(barrier, 1)

    s_rdma = pltpu.make_async_remote_copy(
        src_ref=s_ref, dst_ref=s_nbr, send_sem=s_send, recv_sem=s_recv,
        device_id=nbr, device_id_type=MESH)
    s_rdma.start()
    s_rdma.wait()
    recip_ref[...] = 1.0 / (s_ref[...] + s_nbr[...])

    def tile_rdma(i):
        return pltpu.make_async_remote_copy(
            src_ref=buf,
            dst_ref=recv_ref.at[pl.ds(i * CM, CM), :],
            send_sem=send_sems.at[i], recv_sem=recv_sems.at[i],
            device_id=nbr, device_id_type=MESH)

    for my_val in (0, 1):
        @pl.when(my == my_val)
        def _(my_val=my_val):
            my_col = my_val * NV
            nbr_col = (1 - my_val) * NV

            def process_recv(j):
                rows = pl.ds(j * CM, CM)
                tile_rdma(j).wait_recv()
                ld = pltpu.make_async_copy(
                    recv_ref.at[rows, :], rbuf, rload_sem)
                ld.start()
                ld.wait()
                st_buf[...] = rbuf[...].astype(jnp.float32) * recip_ref[rows]
                st = pltpu.make_async_copy(
                    st_buf, out_ref.at[rows, pl.ds(nbr_col, NV)],
                    store_sems.at[j])
                st.start()
                st.wait()

            for i in range(CBLK):
                rows = pl.ds(i * CM, CM)
                ld = pltpu.make_async_copy(e_ref.at[rows, :], buf, load_sem)
                ld.start()
                ld.wait()
                rdma = tile_rdma(i)
                rdma.start()
                st_buf[...] = buf[...].astype(jnp.float32) * recip_ref[rows]
                st = pltpu.make_async_copy(
                    st_buf, out_ref.at[rows, pl.ds(my_col, NV)],
                    store_sems.at[i])
                st.start()
                st.wait()
                if i >= 1:
                    process_recv(i - 1)
                rdma.wait_send()

            process_recv(CBLK - 1)


def _comm_assemble(e, s):
    out, _ = pl.pallas_call(
        _comm_body,
        in_specs=[
            pl.BlockSpec(memory_space=pl.ANY),
            pl.BlockSpec(memory_space=pltpu.MemorySpace.VMEM),
        ],
        out_specs=[
            pl.BlockSpec(memory_space=pl.ANY),
            pl.BlockSpec(memory_space=pl.ANY),
        ],
        out_shape=[
            jax.ShapeDtypeStruct((M, 2 * NV), jnp.float32),
            jax.ShapeDtypeStruct((M, NV), jnp.bfloat16),
        ],
        scratch_shapes=[
            pltpu.MemorySpace.VMEM((CM, NV), jnp.bfloat16),
            pltpu.MemorySpace.VMEM((CM, NV), jnp.bfloat16),
            pltpu.MemorySpace.VMEM((CM, NV), jnp.float32),
            pltpu.MemorySpace.VMEM((M, 1), jnp.float32),
            pltpu.MemorySpace.VMEM((M, 1), jnp.float32),
            pltpu.SemaphoreType.DMA((CBLK,)),
            pltpu.SemaphoreType.DMA((CBLK,)),
            pltpu.SemaphoreType.DMA((CBLK,)),
            pltpu.SemaphoreType.DMA,
            pltpu.SemaphoreType.DMA,
            pltpu.SemaphoreType.DMA,
            pltpu.SemaphoreType.DMA,
        ],
        compiler_params=pltpu.CompilerParams(collective_id=0),
    )(e, s)
    return out


def kernel(x, W):
    x16 = x.astype(jnp.bfloat16)
    W16 = W.astype(jnp.bfloat16)
    e, s_part = _gemm_exp(x16, W16)
    return _comm_assemble(e, s_part[:, 0:1])


# device time: 727761 ns/iter; 1.7025x vs baseline; 1.0166x over previous
import jax
import jax.numpy as jnp
from jax import lax
from jax.experimental import pallas as pl
from jax.experimental.pallas import tpu as pltpu

M = 2048
K = 4096
NV = 8192

BM = 1024
BN = 512
MBLK = M // BM
NBLK = NV // BN

CM = 256
CBLK = M // CM

MESH = pl.DeviceIdType.MESH


def _gemm_body(x_ref, w_ref, e_ref, s_ref):
    acc = jnp.dot(x_ref[...], w_ref[...], preferred_element_type=jnp.float32)
    e = jnp.exp(acc)
    e_ref[...] = e.astype(jnp.bfloat16)

    @pl.when(pl.program_id(1) == 0)
    def _():
        s_ref[...] = jnp.zeros_like(s_ref)

    s_ref[:, 0:1] = s_ref[:, 0:1] + jnp.sum(e, axis=1, keepdims=True)


def _gemm_exp(x16, W16):
    return pl.pallas_call(
        _gemm_body,
        grid=(MBLK, NBLK),
        in_specs=[
            pl.BlockSpec((BM, K), lambda i, j: (i, 0)),
            pl.BlockSpec((K, BN), lambda i, j: (0, j)),
        ],
        out_specs=[
            pl.BlockSpec((BM, BN), lambda i, j: (i, j)),
            pl.BlockSpec((BM, 128), lambda i, j: (i, 0)),
        ],
        out_shape=[
            jax.ShapeDtypeStruct((M, NV), jnp.bfloat16),
            jax.ShapeDtypeStruct((M, 128), jnp.float32),
        ],
    )(x16, W16)


def _comm_body(e_ref, s_ref, out_ref, recv_ref, buf, rbuf, st_buf,
               s_nbr, recip_ref,
               send_sems, recv_sems, store_sems, load_sem, rload_sem,
               s_send, s_recv):
    mx = lax.axis_index("x")
    my = lax.axis_index("y")
    mz = lax.axis_index("z")
    nbr = (mx, 1 - my, mz)

    barrier = pltpu.get_barrier_semaphore()
    pl.semaphore_signal(barrier, inc=1, device_id=nbr, device_id_type=MESH)
    pl.semaphore_wait(barrier, 1)

    s_rdma = pltpu.make_async_remote_copy(
        src_ref=s_ref, dst_ref=s_nbr, send_sem=s_send, recv_sem=s_recv,
        device_id=nbr, device_id_type=MESH)
    s_rdma.start()
    s_rdma.wait()
    recip_ref[...] = 1.0 / (s_ref[...] + s_nbr[...])

    def tile_rdma(i):
        return pltpu.make_async_remote_copy(
            src_ref=buf,
            dst_ref=recv_ref.at[pl.ds(i * CM, CM), :],
            send_sem=send_sems.at[i], recv_sem=recv_sems.at[i],
            device_id=nbr, device_id_type=MESH)

    for my_val in (0, 1):
        @pl.when(my == my_val)
        def _(my_val=my_val):
            my_col = my_val * NV
            nbr_col = (1 - my_val) * NV

            def process_recv(j):
                rows = pl.ds(j * CM, CM)
                tile_rdma(j).wait_recv()
                ld = pltpu.make_async_copy(
                    recv_ref.at[rows, :], rbuf, rload_sem)
                ld.start()
                ld.wait()
                st_buf[...] = rbuf[...].astype(jnp.float32) * recip_ref[rows]
                st = pltpu.make_async_copy(
                    st_buf, out_ref.at[rows, pl.ds(nbr_col, NV)],
                    store_sems.at[j])
                st.start()
                st.wait()

            for i in range(CBLK):
                rows = pl.ds(i * CM, CM)
                ld = pltpu.make_async_copy(e_ref.at[rows, :], buf, load_sem)
                ld.start()
                ld.wait()
                rdma = tile_rdma(i)
                rdma.start()
                st_buf[...] = buf[...].astype(jnp.float32) * recip_ref[rows]
                st = pltpu.make_async_copy(
                    st_buf, out_ref.at[rows, pl.ds(my_col, NV)],
                    store_sems.at[i])
                st.start()
                st.wait()
                if i >= 1:
                    process_recv(i - 1)
                rdma.wait_send()

            process_recv(CBLK - 1)


def _comm_assemble(e, s):
    out, _ = pl.pallas_call(
        _comm_body,
        in_specs=[
            pl.BlockSpec(memory_space=pl.ANY),
            pl.BlockSpec(memory_space=pltpu.MemorySpace.VMEM),
        ],
        out_specs=[
            pl.BlockSpec(memory_space=pl.ANY),
            pl.BlockSpec(memory_space=pl.ANY),
        ],
        out_shape=[
            jax.ShapeDtypeStruct((M, 2 * NV), jnp.float32),
            jax.ShapeDtypeStruct((M, NV), jnp.bfloat16),
        ],
        scratch_shapes=[
            pltpu.MemorySpace.VMEM((CM, NV), jnp.bfloat16),
            pltpu.MemorySpace.VMEM((CM, NV), jnp.bfloat16),
            pltpu.MemorySpace.VMEM((CM, NV), jnp.float32),
            pltpu.MemorySpace.VMEM((M, 1), jnp.float32),
            pltpu.MemorySpace.VMEM((M, 1), jnp.float32),
            pltpu.SemaphoreType.DMA((CBLK,)),
            pltpu.SemaphoreType.DMA((CBLK,)),
            pltpu.SemaphoreType.DMA((CBLK,)),
            pltpu.SemaphoreType.DMA,
            pltpu.SemaphoreType.DMA,
            pltpu.SemaphoreType.DMA,
            pltpu.SemaphoreType.DMA,
        ],
        compiler_params=pltpu.CompilerParams(collective_id=0),
    )(e, s)
    return out


def kernel(x, W):
    x16 = x.astype(jnp.bfloat16)
    W16 = W.astype(jnp.bfloat16)
    e, s_part = _gemm_exp(x16, W16)
    return _comm_assemble(e, s_part[:, 0:1])
